# initial kernel scaffold (unmeasured)
import jax
import jax.numpy as jnp
from jax import lax
from jax.experimental import pallas as pl
from jax.experimental.pallas import tpu as pltpu


def kernel(
    x,
):
    def body(*refs):
        pass

    out_shape = jax.ShapeDtypeStruct(..., jnp.float32)
    return pl.pallas_call(body, out_shape=out_shape)(...)


# baseline (device time: 11117 ns/iter reference)
import jax
import jax.numpy as jnp
from jax import lax
from jax.experimental import pallas as pl
from jax.experimental.pallas import tpu as pltpu

K = 8
PAD = 128


def kernel(x):
    m, n = x.shape
    dtype = x.dtype
    neg_inf = float("-inf")

    def _topk_into(vals, width, out_width):
        iota = lax.broadcasted_iota(jnp.int32, (m, width), 1)
        out_iota = lax.broadcasted_iota(jnp.int32, (m, out_width), 1)
        out = jnp.full((m, out_width), neg_inf, dtype)
        for k in range(K):
            mx = jnp.max(vals, axis=1, keepdims=True)
            first = jnp.min(
                jnp.where(vals == mx, iota, width), axis=1, keepdims=True
            )
            out = jnp.where(out_iota == k, mx, out)
            vals = jnp.where(iota == first, neg_inf, vals)
        return out

    def body(x_ref, out_ref, cand_ref, send_sem, recv_sem):
        my_x = lax.axis_index("x")
        my_y = lax.axis_index("y")
        nbr = (my_x, 1 - my_y)

        local = _topk_into(x_ref[:, :], n, PAD)
        cand_ref[0, :, :] = local

        barrier_sem = pltpu.get_barrier_semaphore()
        pl.semaphore_signal(
            barrier_sem, inc=1, device_id=nbr,
            device_id_type=pl.DeviceIdType.MESH,
        )
        pl.semaphore_wait(barrier_sem, 1)

        rdma = pltpu.make_async_remote_copy(
            src_ref=cand_ref.at[0],
            dst_ref=cand_ref.at[1],
            send_sem=send_sem,
            recv_sem=recv_sem,
            device_id=nbr,
            device_id_type=pl.DeviceIdType.MESH,
        )
        rdma.start()
        rdma.wait()

        both = jnp.concatenate([local, cand_ref[1, :, :]], axis=1)
        out_ref[:, :] = _topk_into(both, 2 * PAD, K)

    return pl.pallas_call(
        body,
        out_shape=jax.ShapeDtypeStruct((m, K), dtype),
        in_specs=[pl.BlockSpec(memory_space=pltpu.VMEM)],
        out_specs=pl.BlockSpec(memory_space=pltpu.VMEM),
        scratch_shapes=[
            pltpu.VMEM((2, m, PAD), dtype),
            pltpu.SemaphoreType.DMA,
            pltpu.SemaphoreType.DMA,
        ],
        compiler_params=pltpu.CompilerParams(collective_id=0),
    )(x)


# device time: 10267 ns/iter; 1.0828x vs baseline; 1.0828x over previous
import jax
import jax.numpy as jnp
from jax import lax
from jax.experimental import pallas as pl
from jax.experimental.pallas import tpu as pltpu

K = 8
IDX_MASK = 0x3FF
KEY_MIN = -(2**31)


def kernel(x):
    m, n = x.shape
    dtype = x.dtype

    def _pack(vals, width):
        b = lax.bitcast_convert_type(vals, jnp.int32)
        s = jnp.where(b >= 0, b, b ^ 0x7FFFFFFF)
        iota = lax.broadcasted_iota(jnp.int32, (m, width), 1)
        return (s & ~IDX_MASK) | iota

    def _unpack(keys):
        s = keys & ~IDX_MASK
        b = jnp.where(s >= 0, s, s ^ 0x7FFFFFFF)
        return lax.bitcast_convert_type(b, dtype)

    def _topk_keys(keys, width):
        out_iota = lax.broadcasted_iota(jnp.int32, (m, K), 1)
        out = jnp.full((m, K), KEY_MIN, jnp.int32)
        for k in range(K):
            mx = jnp.max(keys, axis=1, keepdims=True)
            out = jnp.where(out_iota == k, mx, out)
            keys = jnp.where(keys == mx, KEY_MIN, keys)
        return out

    def body(x_ref, out_ref, cand_ref, send_sem, recv_sem):
        my_x = lax.axis_index("x")
        my_y = lax.axis_index("y")
        nbr = (my_x, 1 - my_y)

        barrier_sem = pltpu.get_barrier_semaphore()
        pl.semaphore_signal(
            barrier_sem, inc=1, device_id=nbr,
            device_id_type=pl.DeviceIdType.MESH,
        )

        local = _topk_keys(_pack(x_ref[:, :], n), n)
        cand_ref[0, :, :] = local

        pl.semaphore_wait(barrier_sem, 1)

        rdma = pltpu.make_async_remote_copy(
            src_ref=cand_ref.at[0],
            dst_ref=cand_ref.at[1],
            send_sem=send_sem,
            recv_sem=recv_sem,
            device_id=nbr,
            device_id_type=pl.DeviceIdType.MESH,
        )
        rdma.start()
        rdma.wait()

        both = jnp.concatenate([local, cand_ref[1, :, :]], axis=1)
        iota16 = lax.broadcasted_iota(jnp.int32, (m, 2 * K), 1)
        both = (both & ~IDX_MASK) | iota16
        out_ref[:, :] = _unpack(_topk_keys(both, 2 * K))

    return pl.pallas_call(
        body,
        out_shape=jax.ShapeDtypeStruct((m, K), dtype),
        in_specs=[pl.BlockSpec(memory_space=pltpu.VMEM)],
        out_specs=pl.BlockSpec(memory_space=pltpu.VMEM),
        scratch_shapes=[
            pltpu.VMEM((2, m, K), jnp.int32),
            pltpu.SemaphoreType.DMA,
            pltpu.SemaphoreType.DMA,
        ],
        compiler_params=pltpu.CompilerParams(collective_id=0),
    )(x)


# device time: 9781 ns/iter; 1.1366x vs baseline; 1.0497x over previous
import jax
import jax.numpy as jnp
from jax import lax
from jax.experimental import pallas as pl
from jax.experimental.pallas import tpu as pltpu

K = 8
IDX_MASK = 0x3FF
KEY_MIN = -(2**31)
N_HALF = 2


def kernel(x):
    m, n = x.shape
    dtype = x.dtype
    rows = m // N_HALF

    def _pack(vals):
        b = lax.bitcast_convert_type(vals, jnp.int32)
        s = jnp.where(b >= 0, b, b ^ 0x7FFFFFFF)
        iota = lax.broadcasted_iota(jnp.int32, (rows, n), 1)
        return (s & ~IDX_MASK) | iota

    def _unpack(keys):
        s = keys & ~IDX_MASK
        b = jnp.where(s >= 0, s, s ^ 0x7FFFFFFF)
        return lax.bitcast_convert_type(b, dtype)

    def _top8_desc_asc(keys):
        out_iota = lax.broadcasted_iota(jnp.int32, (rows, K), 1)
        desc = jnp.full((rows, K), KEY_MIN, jnp.int32)
        asc = jnp.full((rows, K), KEY_MIN, jnp.int32)
        for k in range(K):
            mx = jnp.max(keys, axis=1, keepdims=True)
            desc = jnp.where(out_iota == k, mx, desc)
            asc = jnp.where(out_iota == K - 1 - k, mx, asc)
            keys = jnp.where(keys == mx, KEY_MIN, keys)
        return desc, asc

    def _merge_top8(desc_mine, asc_theirs):
        out = jnp.maximum(desc_mine, asc_theirs)
        iota8 = lax.broadcasted_iota(jnp.int32, (rows, K), 1)
        for d in (4, 2, 1):
            up = pltpu.roll(out, d, 1)
            down = pltpu.roll(out, K - d, 1)
            hi_lane = (iota8 & d) != 0
            partner = jnp.where(hi_lane, up, down)
            out = jnp.where(
                hi_lane, jnp.minimum(out, partner), jnp.maximum(out, partner)
            )
        return out

    def body(x_ref, out_ref, send_ref, recv_ref, send_sems, recv_sems):
        my_x = lax.axis_index("x")
        my_y = lax.axis_index("y")
        nbr = (my_x, 1 - my_y)

        barrier_sem = pltpu.get_barrier_semaphore()
        pl.semaphore_signal(
            barrier_sem, inc=1, device_id=nbr,
            device_id_type=pl.DeviceIdType.MESH,
        )

        def _swap(h):
            return pltpu.make_async_remote_copy(
                src_ref=send_ref.at[h],
                dst_ref=recv_ref.at[h],
                send_sem=send_sems.at[h],
                recv_sem=recv_sems.at[h],
                device_id=nbr,
                device_id_type=pl.DeviceIdType.MESH,
            )

        descs = []
        rdmas = []
        for h in range(N_HALF):
            desc, asc = _top8_desc_asc(
                _pack(x_ref[pl.ds(h * rows, rows), :])
            )
            descs.append(desc)
            send_ref[h, :, :] = asc
            if h == 0:
                pl.semaphore_wait(barrier_sem, 1)
            rdma = _swap(h)
            rdma.start()
            rdmas.append(rdma)

        for h in range(N_HALF):
            rdmas[h].wait()
            merged = _merge_top8(descs[h], recv_ref[h, :, :])
            out_ref[pl.ds(h * rows, rows), :] = _unpack(merged)

    return pl.pallas_call(
        body,
        out_shape=jax.ShapeDtypeStruct((m, K), dtype),
        in_specs=[pl.BlockSpec(memory_space=pltpu.VMEM)],
        out_specs=pl.BlockSpec(memory_space=pltpu.VMEM),
        scratch_shapes=[
            pltpu.VMEM((N_HALF, rows, K), jnp.int32),
            pltpu.VMEM((N_HALF, rows, K), jnp.int32),
            pltpu.SemaphoreType.DMA((N_HALF,)),
            pltpu.SemaphoreType.DMA((N_HALF,)),
        ],
        compiler_params=pltpu.CompilerParams(collective_id=0),
    )(x)
